# Initial kernel scaffold; baseline (speedup 1.0000x reference)
import jax
import jax.numpy as jnp
from jax.experimental import pallas as pl


def kernel(x, down0_w1, down0_b1, down0_w2, down0_b2, down1_w1, down1_b1, down1_w2, down1_b2, mid_w1, mid_b1, mid_w2, mid_b2, ups0_w, ups0_b, ups1_w, ups1_b, upc0_w1, upc0_b1, upc0_w2, upc0_b2, upc1_w1, upc1_b1, upc1_w2, upc1_b2, proj_w, proj_b):
    raise NotImplementedError("write your pallas kernel here")


# speedup vs baseline: 1.4216x; 1.4216x over previous
import jax
import jax.numpy as jnp
from jax.experimental import pallas as pl
from jax.experimental.pallas import tpu as pltpu


def _zero_border(ref):
    Hp, Wp, C = ref.shape
    z_row = jnp.zeros((1, Wp, C), ref.dtype)
    z_col = jnp.zeros((Hp, 1, C), ref.dtype)
    ref[0:1, :, :] = z_row
    ref[Hp - 1:Hp, :, :] = z_row
    ref[:, 0:1, :] = z_col
    ref[:, Wp - 1:Wp, :] = z_col


def _conv3x3(pad_ref, colw_ref, w_ref, b_ref, H, W, cin):
    for j in range(3):
        colw_ref[:, :, j * cin:(j + 1) * cin] = pad_ref[:, j:j + W, :]
    acc = None
    for kh in range(3):
        lhs = colw_ref[kh:kh + H].reshape(H * W, 3 * cin)
        d = jnp.dot(lhs, w_ref[kh], preferred_element_type=jnp.float32)
        acc = d if acc is None else acc + d
    return jnp.maximum(acc + b_ref[...], 0.0)


def _make_block_kernel(H, W, Ca, Cb, C1, C2, in_pooled, tail):
    has_skip = Cb > 0
    Cin = Ca + Cb

    def body(*args):
        i = 0
        x_ref = args[i]; i += 1
        skip_ref = None
        if has_skip:
            skip_ref = args[i]; i += 1
        w1_ref, b1_ref, w2_ref, b2_ref = args[i:i + 4]; i += 4
        wt_ref = bt_ref = None
        if tail in ("up", "proj"):
            wt_ref, bt_ref = args[i:i + 2]; i += 2
        if tail == "pool":
            o_ref, op_ref = args[i:i + 2]; i += 2
        else:
            o_ref = args[i]; i += 1
        xpad_ref, colw1_ref, hpad_ref, colw2_ref = args[i:i + 4]

        _zero_border(xpad_ref)
        _zero_border(hpad_ref)

        if in_pooled:
            xin = jnp.maximum(x_ref[0][:, :, :Ca], x_ref[0][:, :, Ca:])
        else:
            xin = x_ref[0]
        xpad_ref[1:H + 1, 1:W + 1, 0:Ca] = xin
        if has_skip:
            xpad_ref[1:H + 1, 1:W + 1, Ca:Cin] = skip_ref[0]

        h1 = _conv3x3(xpad_ref, colw1_ref, w1_ref, b1_ref, H, W, Cin)
        hpad_ref[1:H + 1, 1:W + 1, :] = h1.astype(jnp.bfloat16).reshape(H, W, C1)
        h2 = _conv3x3(hpad_ref, colw2_ref, w2_ref, b2_ref, H, W, C1)
        h2b = h2.astype(jnp.bfloat16)

        if tail == "pool":
            o_ref[0] = h2b.reshape(H, W, C2)
            v = h2b.reshape(H // 2, 2, W, C2)
            op_ref[0] = jnp.maximum(v[:, 0], v[:, 1])
        elif tail == "up":
            y = jnp.dot(h2b, wt_ref[...], preferred_element_type=jnp.float32)
            o_ref[0] = (y + bt_ref[...]).astype(jnp.bfloat16)
        else:
            y = jnp.dot(h2b, wt_ref[...], preferred_element_type=jnp.float32)
            o_ref[0] = y + bt_ref[...]

    return body


def _block_call(x, skip, w1, b1, w2, b2, wt, bt, H, W, Ca, Cb, C1, C2,
                in_pooled, tail, cout_tail=None):
    N = x.shape[0]
    Cin = Ca + Cb
    Cx = 2 * Ca if in_pooled else Ca

    inputs = [x]
    in_specs = [pl.BlockSpec((1, H, W, Cx), lambda n: (n, 0, 0, 0))]
    if skip is not None:
        inputs.append(skip)
        in_specs.append(pl.BlockSpec((1, H, W, Cb), lambda n: (n, 0, 0, 0)))
    inputs += [w1, b1, w2, b2]
    in_specs += [
        pl.BlockSpec(w1.shape, lambda n: (0, 0, 0)),
        pl.BlockSpec(b1.shape, lambda n: (0, 0)),
        pl.BlockSpec(w2.shape, lambda n: (0, 0, 0)),
        pl.BlockSpec(b2.shape, lambda n: (0, 0)),
    ]
    if tail in ("up", "proj"):
        inputs += [wt, bt]
        in_specs += [
            pl.BlockSpec(wt.shape, lambda n: (0, 0)),
            pl.BlockSpec(bt.shape, lambda n: (0, 0)),
        ]

    if tail == "pool":
        out_shape = [
            jax.ShapeDtypeStruct((N, H, W, C2), jnp.bfloat16),
            jax.ShapeDtypeStruct((N, H // 2, W, C2), jnp.bfloat16),
        ]
        out_specs = [
            pl.BlockSpec((1, H, W, C2), lambda n: (n, 0, 0, 0)),
            pl.BlockSpec((1, H // 2, W, C2), lambda n: (n, 0, 0, 0)),
        ]
    elif tail == "up":
        out_shape = jax.ShapeDtypeStruct((N, H * W, cout_tail), jnp.bfloat16)
        out_specs = pl.BlockSpec((1, H * W, cout_tail), lambda n: (n, 0, 0))
    else:
        out_shape = jax.ShapeDtypeStruct((N, H * W, cout_tail), jnp.float32)
        out_specs = pl.BlockSpec((1, H * W, cout_tail), lambda n: (n, 0, 0))

    scratch = [
        pltpu.VMEM((H + 2, W + 2, Cin), jnp.bfloat16),
        pltpu.VMEM((H + 2, W, 3 * Cin), jnp.bfloat16),
        pltpu.VMEM((H + 2, W + 2, C1), jnp.bfloat16),
        pltpu.VMEM((H + 2, W, 3 * C1), jnp.bfloat16),
    ]

    return pl.pallas_call(
        _make_block_kernel(H, W, Ca, Cb, C1, C2, in_pooled, tail),
        out_shape=out_shape,
        grid=(N,),
        in_specs=in_specs,
        out_specs=out_specs,
        scratch_shapes=scratch,
        compiler_params=pltpu.CompilerParams(
            dimension_semantics=("parallel",)),
    )(*inputs)


def _prep_w(w):
    kh, kw, cin, cout = w.shape
    return w.reshape(kh, kw * cin, cout).astype(jnp.bfloat16)


def _prep_up(w, b):
    cin, cout = w.shape[2], w.shape[3]
    wm = jnp.transpose(w, (2, 0, 1, 3)).reshape(cin, 4 * cout).astype(jnp.bfloat16)
    bm = jnp.tile(b.reshape(1, cout), (1, 4))
    return wm, bm


def _interleave(y, N, H, W, C):
    y = y.reshape(N, H, W, 2, 2, C).transpose(0, 1, 3, 2, 4, 5)
    return y.reshape(N, 2 * H, 2 * W, C)


def kernel(x,
           down0_w1, down0_b1, down0_w2, down0_b2,
           down1_w1, down1_b1, down1_w2, down1_b2,
           mid_w1, mid_b1, mid_w2, mid_b2,
           ups0_w, ups0_b, ups1_w, ups1_b,
           upc0_w1, upc0_b1, upc0_w2, upc0_b2,
           upc1_w1, upc1_b1, upc1_w2, upc1_b2,
           proj_w, proj_b):
    N = x.shape[0]
    xh = jnp.transpose(x, (0, 2, 3, 1)).astype(jnp.bfloat16)

    d0w1, d0w2 = _prep_w(down0_w1), _prep_w(down0_w2)
    d1w1, d1w2 = _prep_w(down1_w1), _prep_w(down1_w2)
    mw1, mw2 = _prep_w(mid_w1), _prep_w(mid_w2)
    u0w1, u0w2 = _prep_w(upc0_w1), _prep_w(upc0_w2)
    u1w1, u1w2 = _prep_w(upc1_w1), _prep_w(upc1_w2)
    up0w, up0b = _prep_up(ups0_w, ups0_b)
    up1w, up1b = _prep_up(ups1_w, ups1_b)
    pw = proj_w.astype(jnp.bfloat16)

    skip0, hp0 = _block_call(xh, None, d0w1, down0_b1, d0w2, down0_b2,
                             None, None, 64, 64, 3, 0, 64, 64,
                             in_pooled=False, tail="pool")
    hp0 = hp0.reshape(N, 32, 32, 128)
    skip1, hp1 = _block_call(hp0, None, d1w1, down1_b1, d1w2, down1_b2,
                             None, None, 32, 32, 64, 0, 128, 128,
                             in_pooled=True, tail="pool")
    hp1 = hp1.reshape(N, 16, 16, 256)
    y0 = _block_call(hp1, None, mw1, mid_b1, mw2, mid_b2, up0w, up0b,
                     16, 16, 128, 0, 256, 256,
                     in_pooled=True, tail="up", cout_tail=512)
    y0 = _interleave(y0, N, 16, 16, 128)
    y1 = _block_call(y0, skip1, u0w1, upc0_b1, u0w2, upc0_b2, up1w, up1b,
                     32, 32, 128, 128, 128, 128,
                     in_pooled=False, tail="up", cout_tail=256)
    y1 = _interleave(y1, N, 32, 32, 64)
    out = _block_call(y1, skip0, u1w1, upc1_b1, u1w2, upc1_b2, pw, proj_b,
                      64, 64, 64, 64, 64, 64,
                      in_pooled=False, tail="proj", cout_tail=3)
    out = jnp.transpose(out, (0, 2, 1)).reshape(N, 3, 64, 64)
    return out


# speedup vs baseline: 1.5428x; 1.0853x over previous
import jax
import jax.numpy as jnp
from jax.experimental import pallas as pl
from jax.experimental.pallas import tpu as pltpu


def _zero_border(ref):
    Hp, Wp, C = ref.shape
    z_row = jnp.zeros((1, Wp, C), ref.dtype)
    z_col = jnp.zeros((Hp, 1, C), ref.dtype)
    ref[0:1, :, :] = z_row
    ref[Hp - 1:Hp, :, :] = z_row
    ref[:, 0:1, :] = z_col
    ref[:, Wp - 1:Wp, :] = z_col


def _conv3x3(pad_ref, colw_ref, w_ref, b_ref, H, W, cin):
    for j in range(3):
        colw_ref[:, :, j * cin:(j + 1) * cin] = pad_ref[:, j:j + W, :]
    acc = None
    for kh in range(3):
        lhs = colw_ref[kh:kh + H].reshape(H * W, 3 * cin)
        d = jnp.dot(lhs, w_ref[kh], preferred_element_type=jnp.float32)
        acc = d if acc is None else acc + d
    return jnp.maximum(acc + b_ref[...], 0.0)


def _make_block_kernel(H, W, Ca, Cb, C1, C2, in_pooled, tail):
    has_skip = Cb > 0
    Cin = Ca + Cb

    def body(*args):
        i = 0
        x_ref = args[i]; i += 1
        skip_ref = None
        if has_skip:
            skip_ref = args[i]; i += 1
        w1_ref, b1_ref, w2_ref, b2_ref = args[i:i + 4]; i += 4
        wt_ref = bt_ref = None
        if tail in ("up", "proj"):
            wt_ref, bt_ref = args[i:i + 2]; i += 2
        if tail == "pool":
            o_ref, op_ref = args[i:i + 2]; i += 2
        else:
            o_ref = args[i]; i += 1
        xpad_ref, colw1_ref, hpad_ref, colw2_ref = args[i:i + 4]; i += 4
        up_ref = args[i] if tail == "up" else None

        _zero_border(xpad_ref)
        _zero_border(hpad_ref)

        if in_pooled:
            xin = jnp.maximum(x_ref[0][:, :, :Ca], x_ref[0][:, :, Ca:])
        else:
            xin = x_ref[0]
        xpad_ref[1:H + 1, 1:W + 1, 0:Ca] = xin
        if has_skip:
            xpad_ref[1:H + 1, 1:W + 1, Ca:Cin] = skip_ref[0]

        h1 = _conv3x3(xpad_ref, colw1_ref, w1_ref, b1_ref, H, W, Cin)
        hpad_ref[1:H + 1, 1:W + 1, :] = h1.astype(jnp.bfloat16).reshape(H, W, C1)
        h2 = _conv3x3(hpad_ref, colw2_ref, w2_ref, b2_ref, H, W, C1)
        h2b = h2.astype(jnp.bfloat16)

        if tail == "pool":
            o_ref[0] = h2b.reshape(H, W, C2)
            v = h2b.reshape(H // 2, 2, W, C2)
            op_ref[0] = jnp.maximum(v[:, 0], v[:, 1])
        elif tail == "up":
            y = jnp.dot(h2b, wt_ref[...], preferred_element_type=jnp.float32)
            y = y + bt_ref[...]
            Co = y.shape[-1] // 4
            for dh in range(2):
                for dw in range(2):
                    k = dh * 2 + dw
                    part = y[:, k * Co:(k + 1) * Co].reshape(H, W, Co)
                    up_ref[pl.ds(dh, H, 2), pl.ds(dw, W, 2), :] = part
            o_ref[0] = up_ref[...].astype(jnp.bfloat16)
        else:
            yt = jax.lax.dot_general(
                wt_ref[...], h2b, (((0,), (1,)), ((), ())),
                preferred_element_type=jnp.float32)
            o_ref[0] = yt + bt_ref[...]

    return body


def _block_call(x, skip, w1, b1, w2, b2, wt, bt, H, W, Ca, Cb, C1, C2,
                in_pooled, tail, cout_tail=None):
    N = x.shape[0]
    Cin = Ca + Cb
    Cx = 2 * Ca if in_pooled else Ca

    inputs = [x]
    in_specs = [pl.BlockSpec((1, H, W, Cx), lambda n: (n, 0, 0, 0))]
    if skip is not None:
        inputs.append(skip)
        in_specs.append(pl.BlockSpec((1, H, W, Cb), lambda n: (n, 0, 0, 0)))
    inputs += [w1, b1, w2, b2]
    in_specs += [
        pl.BlockSpec(w1.shape, lambda n: (0, 0, 0)),
        pl.BlockSpec(b1.shape, lambda n: (0, 0)),
        pl.BlockSpec(w2.shape, lambda n: (0, 0, 0)),
        pl.BlockSpec(b2.shape, lambda n: (0, 0)),
    ]
    if tail in ("up", "proj"):
        inputs += [wt, bt]
        in_specs += [
            pl.BlockSpec(wt.shape, lambda n: (0, 0)),
            pl.BlockSpec(bt.shape, lambda n: (0, 0)),
        ]

    if tail == "pool":
        out_shape = [
            jax.ShapeDtypeStruct((N, H, W, C2), jnp.bfloat16),
            jax.ShapeDtypeStruct((N, H // 2, W, C2), jnp.bfloat16),
        ]
        out_specs = [
            pl.BlockSpec((1, H, W, C2), lambda n: (n, 0, 0, 0)),
            pl.BlockSpec((1, H // 2, W, C2), lambda n: (n, 0, 0, 0)),
        ]
    elif tail == "up":
        Co = cout_tail // 4
        out_shape = jax.ShapeDtypeStruct((N, 2 * H, 2 * W, Co), jnp.bfloat16)
        out_specs = pl.BlockSpec((1, 2 * H, 2 * W, Co), lambda n: (n, 0, 0, 0))
    else:
        out_shape = jax.ShapeDtypeStruct((N, cout_tail, H * W), jnp.float32)
        out_specs = pl.BlockSpec((1, cout_tail, H * W), lambda n: (n, 0, 0))

    scratch = [
        pltpu.VMEM((H + 2, W + 2, Cin), jnp.bfloat16),
        pltpu.VMEM((H + 2, W, 3 * Cin), jnp.bfloat16),
        pltpu.VMEM((H + 2, W + 2, C1), jnp.bfloat16),
        pltpu.VMEM((H + 2, W, 3 * C1), jnp.bfloat16),
    ]
    if tail == "up":
        scratch.append(pltpu.VMEM((2 * H, 2 * W, cout_tail // 4), jnp.float32))

    return pl.pallas_call(
        _make_block_kernel(H, W, Ca, Cb, C1, C2, in_pooled, tail),
        out_shape=out_shape,
        grid=(N,),
        in_specs=in_specs,
        out_specs=out_specs,
        scratch_shapes=scratch,
        compiler_params=pltpu.CompilerParams(
            dimension_semantics=("parallel",)),
    )(*inputs)


def _prep_w(w):
    kh, kw, cin, cout = w.shape
    return w.reshape(kh, kw * cin, cout).astype(jnp.bfloat16)


def _prep_up(w, b):
    cin, cout = w.shape[2], w.shape[3]
    wm = jnp.transpose(w, (2, 0, 1, 3)).reshape(cin, 4 * cout).astype(jnp.bfloat16)
    bm = jnp.tile(b.reshape(1, cout), (1, 4))
    return wm, bm


def kernel(x,
           down0_w1, down0_b1, down0_w2, down0_b2,
           down1_w1, down1_b1, down1_w2, down1_b2,
           mid_w1, mid_b1, mid_w2, mid_b2,
           ups0_w, ups0_b, ups1_w, ups1_b,
           upc0_w1, upc0_b1, upc0_w2, upc0_b2,
           upc1_w1, upc1_b1, upc1_w2, upc1_b2,
           proj_w, proj_b):
    N = x.shape[0]
    xh = jnp.transpose(x, (0, 2, 3, 1)).astype(jnp.bfloat16)

    d0w1, d0w2 = _prep_w(down0_w1), _prep_w(down0_w2)
    d1w1, d1w2 = _prep_w(down1_w1), _prep_w(down1_w2)
    mw1, mw2 = _prep_w(mid_w1), _prep_w(mid_w2)
    u0w1, u0w2 = _prep_w(upc0_w1), _prep_w(upc0_w2)
    u1w1, u1w2 = _prep_w(upc1_w1), _prep_w(upc1_w2)
    up0w, up0b = _prep_up(ups0_w, ups0_b)
    up1w, up1b = _prep_up(ups1_w, ups1_b)
    pw = proj_w.astype(jnp.bfloat16)
    pbt = proj_b.reshape(3, 1)

    skip0, hp0 = _block_call(xh, None, d0w1, down0_b1, d0w2, down0_b2,
                             None, None, 64, 64, 3, 0, 64, 64,
                             in_pooled=False, tail="pool")
    hp0 = hp0.reshape(N, 32, 32, 128)
    skip1, hp1 = _block_call(hp0, None, d1w1, down1_b1, d1w2, down1_b2,
                             None, None, 32, 32, 64, 0, 128, 128,
                             in_pooled=True, tail="pool")
    hp1 = hp1.reshape(N, 16, 16, 256)
    y0 = _block_call(hp1, None, mw1, mid_b1, mw2, mid_b2, up0w, up0b,
                     16, 16, 128, 0, 256, 256,
                     in_pooled=True, tail="up", cout_tail=512)
    y1 = _block_call(y0, skip1, u0w1, upc0_b1, u0w2, upc0_b2, up1w, up1b,
                     32, 32, 128, 128, 128, 128,
                     in_pooled=False, tail="up", cout_tail=256)
    out = _block_call(y1, skip0, u1w1, upc1_b1, u1w2, upc1_b2, pw, pbt,
                      64, 64, 64, 64, 64, 64,
                      in_pooled=False, tail="proj", cout_tail=3)
    return out.reshape(N, 3, 64, 64)


# speedup vs baseline: 1.6498x; 1.0694x over previous
import jax
import jax.numpy as jnp
from jax.experimental import pallas as pl
from jax.experimental.pallas import tpu as pltpu


_LP = 8


def _zero_border(ref, H, W):
    Hp, Wp, C = ref.shape
    z_row = jnp.zeros((1, Wp, C), ref.dtype)
    z_col = jnp.zeros((Hp, 1, C), ref.dtype)
    ref[0:1, :, :] = z_row
    ref[H + 1:H + 2, :, :] = z_row
    ref[:, _LP - 1:_LP, :] = z_col
    ref[:, _LP + W:_LP + W + 1, :] = z_col


def _conv3x3(pad_ref, colw_ref, w_ref, b_ref, H, W, cin):
    for j in range(3):
        colw_ref[:, :, j * cin:(j + 1) * cin] = \
            pad_ref[:, _LP - 1 + j:_LP - 1 + j + W, :]
    acc = None
    for kh in range(3):
        lhs = colw_ref[kh:kh + H].reshape(H * W, 3 * cin)
        d = jnp.dot(lhs, w_ref[kh], preferred_element_type=jnp.float32)
        acc = d if acc is None else acc + d
    return jnp.maximum(acc + b_ref[...], 0.0)


def _make_block_kernel(H, W, Ca, Cb, C1, C2, in_pooled, tail):
    has_skip = Cb > 0
    Cin = Ca + Cb

    def body(*args):
        i = 0
        x_ref = args[i]; i += 1
        skip_ref = None
        if has_skip:
            skip_ref = args[i]; i += 1
        w1_ref, b1_ref, w2_ref, b2_ref = args[i:i + 4]; i += 4
        wt_ref = bt_ref = None
        if tail in ("up", "proj"):
            wt_ref, bt_ref = args[i:i + 2]; i += 2
        if tail == "pool":
            o_ref, op_ref = args[i:i + 2]; i += 2
        else:
            o_ref = args[i]; i += 1
        xpad_ref, colw1_ref, hpad_ref, colw2_ref = args[i:i + 4]; i += 4
        up_ref = args[i] if tail == "up" else None

        _zero_border(xpad_ref, H, W)
        _zero_border(hpad_ref, H, W)

        if in_pooled:
            xin = jnp.maximum(x_ref[0][:, :, :Ca], x_ref[0][:, :, Ca:])
        else:
            xin = x_ref[0]
        xpad_ref[1:H + 1, _LP:_LP + W, 0:Ca] = xin
        if has_skip:
            xpad_ref[1:H + 1, _LP:_LP + W, Ca:Cin] = skip_ref[0]

        h1 = _conv3x3(xpad_ref, colw1_ref, w1_ref, b1_ref, H, W, Cin)
        hpad_ref[1:H + 1, _LP:_LP + W, :] = \
            h1.astype(jnp.bfloat16).reshape(H, W, C1)
        h2 = _conv3x3(hpad_ref, colw2_ref, w2_ref, b2_ref, H, W, C1)
        h2b = h2.astype(jnp.bfloat16)

        if tail == "pool":
            o_ref[0] = h2b.reshape(H, W, C2)
            v = h2b.reshape(H // 2, 2, W, C2)
            op_ref[0] = jnp.maximum(v[:, 0], v[:, 1])
        elif tail == "up":
            y = jnp.dot(h2b, wt_ref[...], preferred_element_type=jnp.float32)
            y = y + bt_ref[...]
            Co = y.shape[-1] // 4
            for dh in range(2):
                for dw in range(2):
                    k = dh * 2 + dw
                    part = y[:, k * Co:(k + 1) * Co].reshape(H, W, Co)
                    up_ref[pl.ds(dh, H, 2), pl.ds(dw, W, 2), :] = part
            o_ref[0] = up_ref[...].astype(jnp.bfloat16)
        else:
            yt = jax.lax.dot_general(
                wt_ref[...], h2b, (((0,), (1,)), ((), ())),
                preferred_element_type=jnp.float32)
            o_ref[0] = yt + bt_ref[...]

    return body


def _block_call(x, skip, w1, b1, w2, b2, wt, bt, H, W, Ca, Cb, C1, C2,
                in_pooled, tail, cout_tail=None):
    N = x.shape[0]
    Cin = Ca + Cb
    Cx = 2 * Ca if in_pooled else Ca

    inputs = [x]
    in_specs = [pl.BlockSpec((1, H, W, Cx), lambda n: (n, 0, 0, 0))]
    if skip is not None:
        inputs.append(skip)
        in_specs.append(pl.BlockSpec((1, H, W, Cb), lambda n: (n, 0, 0, 0)))
    inputs += [w1, b1, w2, b2]
    in_specs += [
        pl.BlockSpec(w1.shape, lambda n: (0, 0, 0)),
        pl.BlockSpec(b1.shape, lambda n: (0, 0)),
        pl.BlockSpec(w2.shape, lambda n: (0, 0, 0)),
        pl.BlockSpec(b2.shape, lambda n: (0, 0)),
    ]
    if tail in ("up", "proj"):
        inputs += [wt, bt]
        in_specs += [
            pl.BlockSpec(wt.shape, lambda n: (0, 0)),
            pl.BlockSpec(bt.shape, lambda n: (0, 0)),
        ]

    if tail == "pool":
        out_shape = [
            jax.ShapeDtypeStruct((N, H, W, C2), jnp.bfloat16),
            jax.ShapeDtypeStruct((N, H // 2, W, C2), jnp.bfloat16),
        ]
        out_specs = [
            pl.BlockSpec((1, H, W, C2), lambda n: (n, 0, 0, 0)),
            pl.BlockSpec((1, H // 2, W, C2), lambda n: (n, 0, 0, 0)),
        ]
    elif tail == "up":
        Co = cout_tail // 4
        out_shape = jax.ShapeDtypeStruct((N, 2 * H, 2 * W, Co), jnp.bfloat16)
        out_specs = pl.BlockSpec((1, 2 * H, 2 * W, Co), lambda n: (n, 0, 0, 0))
    else:
        out_shape = jax.ShapeDtypeStruct((N, cout_tail, H * W), jnp.float32)
        out_specs = pl.BlockSpec((1, cout_tail, H * W), lambda n: (n, 0, 0))

    scratch = [
        pltpu.VMEM((H + 2, W + _LP + 2, Cin), jnp.bfloat16),
        pltpu.VMEM((H + 2, W, 3 * Cin), jnp.bfloat16),
        pltpu.VMEM((H + 2, W + _LP + 2, C1), jnp.bfloat16),
        pltpu.VMEM((H + 2, W, 3 * C1), jnp.bfloat16),
    ]
    if tail == "up":
        scratch.append(pltpu.VMEM((2 * H, 2 * W, cout_tail // 4), jnp.float32))

    return pl.pallas_call(
        _make_block_kernel(H, W, Ca, Cb, C1, C2, in_pooled, tail),
        out_shape=out_shape,
        grid=(N,),
        in_specs=in_specs,
        out_specs=out_specs,
        scratch_shapes=scratch,
        compiler_params=pltpu.CompilerParams(
            dimension_semantics=("parallel",)),
    )(*inputs)


def _prep_w(w):
    kh, kw, cin, cout = w.shape
    return w.reshape(kh, kw * cin, cout).astype(jnp.bfloat16)


def _prep_up(w, b):
    cin, cout = w.shape[2], w.shape[3]
    wm = jnp.transpose(w, (2, 0, 1, 3)).reshape(cin, 4 * cout).astype(jnp.bfloat16)
    bm = jnp.tile(b.reshape(1, cout), (1, 4))
    return wm, bm


def kernel(x,
           down0_w1, down0_b1, down0_w2, down0_b2,
           down1_w1, down1_b1, down1_w2, down1_b2,
           mid_w1, mid_b1, mid_w2, mid_b2,
           ups0_w, ups0_b, ups1_w, ups1_b,
           upc0_w1, upc0_b1, upc0_w2, upc0_b2,
           upc1_w1, upc1_b1, upc1_w2, upc1_b2,
           proj_w, proj_b):
    N = x.shape[0]
    xh = jnp.transpose(x, (0, 2, 3, 1)).astype(jnp.bfloat16)

    d0w1, d0w2 = _prep_w(down0_w1), _prep_w(down0_w2)
    d1w1, d1w2 = _prep_w(down1_w1), _prep_w(down1_w2)
    mw1, mw2 = _prep_w(mid_w1), _prep_w(mid_w2)
    u0w1, u0w2 = _prep_w(upc0_w1), _prep_w(upc0_w2)
    u1w1, u1w2 = _prep_w(upc1_w1), _prep_w(upc1_w2)
    up0w, up0b = _prep_up(ups0_w, ups0_b)
    up1w, up1b = _prep_up(ups1_w, ups1_b)
    pw = proj_w.astype(jnp.bfloat16)
    pbt = proj_b.reshape(3, 1)

    skip0, hp0 = _block_call(xh, None, d0w1, down0_b1, d0w2, down0_b2,
                             None, None, 64, 64, 3, 0, 64, 64,
                             in_pooled=False, tail="pool")
    hp0 = hp0.reshape(N, 32, 32, 128)
    skip1, hp1 = _block_call(hp0, None, d1w1, down1_b1, d1w2, down1_b2,
                             None, None, 32, 32, 64, 0, 128, 128,
                             in_pooled=True, tail="pool")
    hp1 = hp1.reshape(N, 16, 16, 256)
    y0 = _block_call(hp1, None, mw1, mid_b1, mw2, mid_b2, up0w, up0b,
                     16, 16, 128, 0, 256, 256,
                     in_pooled=True, tail="up", cout_tail=512)
    y1 = _block_call(y0, skip1, u0w1, upc0_b1, u0w2, upc0_b2, up1w, up1b,
                     32, 32, 128, 128, 128, 128,
                     in_pooled=False, tail="up", cout_tail=256)
    out = _block_call(y1, skip0, u1w1, upc1_b1, u1w2, upc1_b2, pw, pbt,
                      64, 64, 64, 64, 64, 64,
                      in_pooled=False, tail="proj", cout_tail=3)
    return out.reshape(N, 3, 64, 64)
